# baseline (speedup 1.0000x reference)
import functools

import jax
import jax.numpy as jnp
from jax.experimental import pallas as pl
import jax.experimental.pallas.tpu as pltpu


def _body(T, HS, q_ref, x_ref, len_ref, w1q_ref, w1k_ref, w1qk_ref, b1_ref,
          w2t_ref, b2_ref, wd_ref, wih_ref, bih_ref, whh_ref, bhh_ref,
          out_ref, qp_scr, s_scr, gi_scr):
    f32 = jnp.float32
    bf16 = jnp.bfloat16
    j = pl.program_id(1)

    @pl.when(j == 0)
    def _():
        qp_scr[...] = (jnp.dot(q_ref[...], w1q_ref[...],
                               preferred_element_type=f32) + b1_ref[...])

    @pl.when(j < T)
    def _():
        xb = x_ref[...].astype(bf16)
        qx = q_ref[...] * xb
        z1 = (qp_scr[...]
              + jnp.dot(xb, w1k_ref[...], preferred_element_type=f32)
              + jnp.dot(qx, w1qk_ref[...], preferred_element_type=f32))
        th1 = jnp.tanh(z1)
        th2 = jnp.tanh(jnp.dot(th1.astype(bf16), w2t_ref[...],
                               preferred_element_type=f32) + b2_ref[...])
        s = jnp.sum(th2 * wd_ref[...], axis=1, keepdims=True)
        s_scr[pl.ds(j, 1), :] = jnp.transpose(s)
        gi = jnp.dot(xb, wih_ref[...], preferred_element_type=f32) \
            + bih_ref[...]
        gi_scr[j] = gi.astype(bf16)

    @pl.when(j == T)
    def _():
        Bb = out_ref.shape[0]
        sT = s_scr[...]
        validT = (jax.lax.broadcasted_iota(jnp.int32, (T, Bb), 0)
                  < len_ref[...])
        sT = jnp.where(validT, sT, f32(-8.8e7))
        sT = sT - jnp.max(sT, axis=0, keepdims=True)
        e = jnp.exp(sT)
        att = jnp.transpose(e / jnp.sum(e, axis=0, keepdims=True))

        whh = whh_ref[...]
        bhh = bhh_ref[...]
        h = jnp.zeros((Bb, HS), f32)
        for t in range(T):
            gi = gi_scr[t].astype(f32)
            gh = (jnp.dot(h.astype(bf16), whh, preferred_element_type=f32)
                  + bhh)
            thr = jnp.tanh(gi[:, :HS] + gh[:, :HS])
            n = jnp.tanh(gi[:, HS:] + gh[:, HS:] * (1.0 + thr))
            a = att[:, t:t + 1]
            h = h + a * (n - h)
        out_ref[...] = h


def kernel(queries, keys, keys_length, W1, b1, W2, b2, Wd, bd, W_ih, W_hh,
           b_ih, b_hh):
    B, T, H = keys.shape
    HS = W_hh.shape[1]
    Bb = 1024
    while B % Bb:
        Bb //= 2

    f32 = jnp.float32
    bf16 = jnp.bfloat16
    keys2 = keys.reshape(B, T * H)
    qb = queries.astype(bf16)
    lenT = jnp.maximum(keys_length.astype(jnp.int32), 1).reshape(1, B)

    w1a, w1b, w1c, w1d = jnp.split(W1, 4, axis=1)
    w1q = (0.5 * (w1a + w1c)).T.astype(bf16)
    w1k = (0.5 * (w1b - w1c)).T.astype(bf16)
    w1qk = (0.5 * w1d).T.astype(bf16)
    b1r = (0.5 * b1).reshape(1, -1)
    w2t = (0.25 * W2.T).astype(bf16)
    b2r = (0.5 * b2 + 0.25 * jnp.sum(W2, axis=1)).reshape(1, -1)
    wd_s = (0.5 * Wd / jnp.sqrt(jnp.float32(H))).reshape(1, -1)

    wih2 = jnp.concatenate([0.5 * W_ih[:HS], W_ih[2 * HS:]],
                           axis=0).T.astype(bf16)
    whh2 = (0.5 * jnp.concatenate([W_hh[:HS], W_hh[2 * HS:]],
                                  axis=0)).T.astype(bf16)
    bih2 = jnp.concatenate([0.5 * (b_ih[:HS] + b_hh[:HS]),
                            b_ih[2 * HS:]]).reshape(1, -1)
    bhh2 = jnp.concatenate([jnp.zeros((HS,), f32),
                            0.5 * b_hh[2 * HS:]]).reshape(1, -1)

    full = lambda i, j: (0, 0)
    out = pl.pallas_call(
        functools.partial(_body, T, HS),
        grid=(B // Bb, T + 1),
        in_specs=[
            pl.BlockSpec((Bb, H), lambda i, j: (i, 0)),
            pl.BlockSpec((Bb, H), lambda i, j: (i, jnp.minimum(j, T - 1))),
            pl.BlockSpec((1, Bb), lambda i, j: (0, i)),
            pl.BlockSpec(w1q.shape, full),
            pl.BlockSpec(w1k.shape, full),
            pl.BlockSpec(w1qk.shape, full),
            pl.BlockSpec(b1r.shape, full),
            pl.BlockSpec(w2t.shape, full),
            pl.BlockSpec(b2r.shape, full),
            pl.BlockSpec(wd_s.shape, full),
            pl.BlockSpec(wih2.shape, full),
            pl.BlockSpec(bih2.shape, full),
            pl.BlockSpec(whh2.shape, full),
            pl.BlockSpec(bhh2.shape, full),
        ],
        out_specs=pl.BlockSpec((Bb, HS), lambda i, j: (i, 0)),
        out_shape=jax.ShapeDtypeStruct((B, HS), f32),
        scratch_shapes=[
            pltpu.VMEM((Bb, w1q.shape[1]), f32),
            pltpu.VMEM((T, Bb), f32),
            pltpu.VMEM((T, Bb, 2 * HS), bf16),
        ],
    )(qb, keys2, lenT, w1q, w1k, w1qk, b1r, w2t, b2r, wd_s, wih2, bih2,
      whh2, bhh2)
    return out


# speedup vs baseline: 2.7725x; 2.7725x over previous
import functools

import jax
import jax.numpy as jnp
from jax.experimental import pallas as pl


def _body(T, HS, CH, q_ref, kt_ref, len_ref, w1q_ref, w1k_ref, w1qk_ref,
          b1_ref, w2t_ref, b2_ref, wd_ref, wih_ref, bih_ref, whh_ref,
          bhh_ref, out_ref):
    f32 = jnp.float32
    bf16 = jnp.bfloat16
    q = q_ref[...]
    Bb, H = q.shape
    w1k = w1k_ref[...]
    w1qk = w1qk_ref[...]
    w2t = w2t_ref[...]
    b2 = b2_ref[...]
    wd = wd_ref[...]
    qpart = jnp.dot(q, w1q_ref[...], preferred_element_type=f32) + b1_ref[...]
    NA = qpart.shape[1]

    cols = []
    for t0 in range(0, T, CH):
        c = min(CH, T - t0)
        x = kt_ref[t0:t0 + c].reshape(c * Bb, H).astype(bf16)
        qt = jnp.broadcast_to(q[None], (c, Bb, H)).reshape(c * Bb, H)
        qpt = jnp.broadcast_to(qpart[None], (c, Bb, NA)).reshape(c * Bb, NA)
        th1 = jnp.tanh(
            qpt
            + jnp.dot(x, w1k, preferred_element_type=f32)
            + jnp.dot(qt * x, w1qk, preferred_element_type=f32))
        th2 = jnp.tanh(jnp.dot(th1.astype(bf16), w2t,
                               preferred_element_type=f32) + b2)
        s = jnp.sum(th2 * wd, axis=1, keepdims=True)
        for t in range(c):
            cols.append(s[t * Bb:(t + 1) * Bb])
    scores = jnp.concatenate(cols, axis=1)

    valid = jax.lax.broadcasted_iota(jnp.int32, (Bb, T), 1) < len_ref[...]
    scores = jnp.where(valid, scores, f32(-8.8e7))
    scores = scores - jnp.max(scores, axis=1, keepdims=True)
    e = jnp.exp(scores)
    att = e / jnp.sum(e, axis=1, keepdims=True)

    wih = wih_ref[...]
    bih = bih_ref[...]
    whh = whh_ref[...]
    bhh = bhh_ref[...]
    h = jnp.zeros((Bb, HS), f32)
    for t in range(T):
        x = kt_ref[t].astype(bf16)
        gi = jnp.dot(x, wih, preferred_element_type=f32) + bih
        gh = jnp.dot(h.astype(bf16), whh, preferred_element_type=f32) + bhh
        thr = jnp.tanh(gi[:, :HS] + gh[:, :HS])
        n = jnp.tanh(gi[:, HS:] + gh[:, HS:] * (1.0 + thr))
        a = att[:, t:t + 1]
        h = h + a * (n - h)
    out_ref[...] = h


def kernel(queries, keys, keys_length, W1, b1, W2, b2, Wd, bd, W_ih, W_hh,
           b_ih, b_hh):
    B, T, H = keys.shape
    HS = W_hh.shape[1]
    Bb = 512
    while B % Bb:
        Bb //= 2

    f32 = jnp.float32
    bf16 = jnp.bfloat16
    kt = jnp.transpose(keys, (1, 0, 2))
    qb = queries.astype(bf16)
    len2 = jnp.maximum(keys_length.astype(jnp.int32), 1).reshape(B, 1)

    w1a, w1b, w1c, w1d = jnp.split(W1, 4, axis=1)
    w1q = (0.5 * (w1a + w1c)).T.astype(bf16)
    w1k = (0.5 * (w1b - w1c)).T.astype(bf16)
    w1qk = (0.5 * w1d).T.astype(bf16)
    b1r = (0.5 * b1).reshape(1, -1)
    w2t = (0.25 * W2.T).astype(bf16)
    b2r = (0.5 * b2 + 0.25 * jnp.sum(W2, axis=1)).reshape(1, -1)
    wd_s = (0.5 * Wd / jnp.sqrt(jnp.float32(H))).reshape(1, -1)

    wih2 = jnp.concatenate([0.5 * W_ih[:HS], W_ih[2 * HS:]],
                           axis=0).T.astype(bf16)
    whh2 = (0.5 * jnp.concatenate([W_hh[:HS], W_hh[2 * HS:]],
                                  axis=0)).T.astype(bf16)
    bih2 = jnp.concatenate([0.5 * (b_ih[:HS] + b_hh[:HS]),
                            b_ih[2 * HS:]]).reshape(1, -1)
    bhh2 = jnp.concatenate([jnp.zeros((HS,), f32),
                            0.5 * b_hh[2 * HS:]]).reshape(1, -1)

    full = lambda i: (0, 0)
    out = pl.pallas_call(
        functools.partial(_body, T, HS, 10),
        grid=(B // Bb,),
        in_specs=[
            pl.BlockSpec((Bb, H), lambda i: (i, 0)),
            pl.BlockSpec((T, Bb, H), lambda i: (0, i, 0)),
            pl.BlockSpec((Bb, 1), lambda i: (i, 0)),
            pl.BlockSpec(w1q.shape, full),
            pl.BlockSpec(w1k.shape, full),
            pl.BlockSpec(w1qk.shape, full),
            pl.BlockSpec(b1r.shape, full),
            pl.BlockSpec(w2t.shape, full),
            pl.BlockSpec(b2r.shape, full),
            pl.BlockSpec(wd_s.shape, full),
            pl.BlockSpec(wih2.shape, full),
            pl.BlockSpec(bih2.shape, full),
            pl.BlockSpec(whh2.shape, full),
            pl.BlockSpec(bhh2.shape, full),
        ],
        out_specs=pl.BlockSpec((Bb, HS), lambda i: (i, 0)),
        out_shape=jax.ShapeDtypeStruct((B, HS), jnp.float32),
    )(qb, kt, len2, w1q, w1k, w1qk, b1r, w2t, b2r, wd_s, wih2, bih2,
      whh2, bhh2)
    return out
